# Initial kernel scaffold; baseline (speedup 1.0000x reference)
import jax
import jax.numpy as jnp
from jax.experimental import pallas as pl


def kernel(user_emb, item_emb, edge_index, edge_weight):
    raise NotImplementedError("write your pallas kernel here")


# speedup vs baseline: 2.5926x; 2.5926x over previous
import functools

import jax
import jax.numpy as jnp
from jax import lax
from jax.experimental import pallas as pl
from jax.experimental.pallas import tpu as pltpu
from jax.experimental.pallas import tpu_sc as plsc

N_USERS = 5000
N = 10000
NP = 10240
D = 256
H = 128
E = 160000
NCORE = 2
NSUB = 16
C = 128
NCHUNK_PT = 80
BLK = 8
EPT = C * NCHUNK_PT
E_PAD = EPT * NSUB
RPT = NP // NSUB
FCH = 64
NFCH = RPT // FCH

_mesh = plsc.VectorSubcoreMesh(core_axis_name="c", subcore_axis_name="s")


@functools.partial(
    pl.kernel,
    out_type=(
        jax.ShapeDtypeStruct((2 * NP, H), jnp.float32),
        jax.ShapeDtypeStruct((2 * NP, H), jnp.float32),
        jax.ShapeDtypeStruct((2 * NP, H), jnp.float32),
    ),
    mesh=_mesh,
    scratch_types=[
        pltpu.VMEM_SHARED((NP, H), jnp.float32),
        pltpu.VMEM((BLK, C), jnp.int32),
        pltpu.VMEM((BLK, C), jnp.int32),
        pltpu.VMEM((BLK, C), jnp.float32),
        pltpu.VMEM((C, H), jnp.float32),
        pltpu.VMEM((FCH, H), jnp.float32),
        pltpu.SemaphoreType.DMA,
    ],
)
def _lightgcn_sc(tab0, srcs, dsts, ws, zeros, out1, out2, outf,
                 acc, srcblk, dstblk, wblk, rows, zbuf, gsem):
    c = lax.axis_index("c")
    s = lax.axis_index("s")
    ebase = s * NCHUNK_PT
    rbase = s * RPT
    off = c * NP

    def run_layer(tab, out):
        pltpu.sync_copy(zeros.at[pl.ds(rbase, RPT)], acc.at[pl.ds(rbase, RPT)])
        plsc.subcore_barrier()

        def _chunk(k, carry):
            @pl.when(lax.rem(k, BLK) == 0)
            def _():
                b0 = pl.multiple_of(ebase + k, BLK)
                pltpu.sync_copy(srcs.at[pl.ds(b0, BLK)], srcblk)
                pltpu.sync_copy(dsts.at[pl.ds(b0, BLK)], dstblk)
                pltpu.sync_copy(ws.at[pl.ds(b0, BLK)], wblk)

                def _ofs(r, cr):
                    for j in range(C // 16):
                        srcblk[r, pl.ds(j * 16, 16)] = (
                            srcblk[r, pl.ds(j * 16, 16)] + off)
                    return cr

                lax.fori_loop(0, BLK, _ofs, 0)

            km = lax.rem(k, BLK)
            pltpu.async_copy(tab.at[srcblk.at[km]], rows, gsem).wait()

            def _grp(g, c2):
                wvec = wblk[km, pl.ds(g * 16, 16)]
                rb = g * 16
                for lane in range(16):
                    w = wvec[lane]
                    r = rb + lane
                    for j in range(H // 16):
                        rows[r, pl.ds(j * 16, 16)] = (
                            rows[r, pl.ds(j * 16, 16)] * w)
                return c2

            lax.fori_loop(0, C // 16, _grp, 0)
            pltpu.sync_copy(rows, acc.at[dstblk.at[km]], add=True)
            return carry

        lax.fori_loop(0, NCHUNK_PT, _chunk, 0)
        plsc.subcore_barrier()
        if out is not None:
            pltpu.sync_copy(acc.at[pl.ds(rbase, RPT)],
                            out.at[pl.ds(c * NP + rbase, RPT)])
            plsc.subcore_barrier()

    run_layer(tab0, out1)
    run_layer(out1, out2)
    run_layer(out2, None)

    for t in range(NFCH):
        r0 = rbase + t * FCH
        g0 = c * NP + r0
        pltpu.sync_copy(acc.at[pl.ds(r0, FCH)], zbuf)
        for srcref in (tab0, out1, out2):
            pltpu.sync_copy(srcref.at[pl.ds(g0, FCH)], rows.at[pl.ds(0, FCH)])
            last = srcref is out2

            def _add(i, carry):
                for j in range(H // 16):
                    v = zbuf[i, pl.ds(j * 16, 16)] + rows[i, pl.ds(j * 16, 16)]
                    if last:
                        v = v * 0.25
                    zbuf[i, pl.ds(j * 16, 16)] = v
                return carry

            lax.fori_loop(0, FCH, _add, 0)
        pltpu.sync_copy(zbuf, outf.at[pl.ds(g0, FCH)])


def kernel(user_emb, item_emb, edge_index, edge_weight):
    all_emb = jnp.concatenate([user_emb, item_emb], axis=0)
    rpad = jnp.zeros((NP - N, H), jnp.float32)
    tab0 = jnp.concatenate(
        [all_emb[:, :H], rpad, all_emb[:, H:], rpad], axis=0)
    dst = edge_index[0]
    src = edge_index[1]
    pad = E_PAD - E
    src_p = jnp.concatenate([src, jnp.zeros((pad,), jnp.int32)]).reshape(-1, C)
    dst_p = jnp.concatenate([dst, jnp.zeros((pad,), jnp.int32)]).reshape(-1, C)
    w_p = jnp.concatenate(
        [edge_weight, jnp.zeros((pad,), jnp.float32)]).reshape(-1, C)
    zeros = jnp.zeros((NP, H), jnp.float32)
    _, _, outf = _lightgcn_sc(tab0, src_p, dst_p, w_p, zeros)
    mean = jnp.concatenate([outf[:N], outf[NP:NP + N]], axis=1)
    return (mean[:N_USERS], mean[N_USERS:])


# speedup vs baseline: 3.2362x; 1.2482x over previous
import functools

import jax
import jax.numpy as jnp
from jax import lax
from jax.experimental import pallas as pl
from jax.experimental.pallas import tpu as pltpu
from jax.experimental.pallas import tpu_sc as plsc

N_USERS = 5000
N = 10000
NP = 10240
D = 256
H = 128
E = 160000
NCORE = 2
NSUB = 16
C = 128
NCHUNK_PT = 80
BLK = 8
NBLK = NCHUNK_PT // BLK
EPT = C * NCHUNK_PT
E_PAD = EPT * NSUB
RPT = NP // NSUB
FCH = 64
NFCH = RPT // FCH

_mesh = plsc.VectorSubcoreMesh(core_axis_name="c", subcore_axis_name="s")


@functools.partial(
    pl.kernel,
    out_type=(
        jax.ShapeDtypeStruct((2 * NP, H), jnp.float32),
        jax.ShapeDtypeStruct((2 * NP, H), jnp.float32),
        jax.ShapeDtypeStruct((2 * NP, H), jnp.float32),
    ),
    mesh=_mesh,
    scratch_types=[
        pltpu.VMEM_SHARED((NP, H), jnp.float32),
        pltpu.VMEM((2 * BLK, C), jnp.int32),
        pltpu.VMEM((BLK, C), jnp.float32),
        pltpu.VMEM((C, H), jnp.float32),
        pltpu.VMEM((C, H), jnp.float32),
        pltpu.VMEM((FCH, H), jnp.float32),
        pltpu.SemaphoreType.DMA,
        pltpu.SemaphoreType.DMA,
    ],
)
def _lightgcn_sc(tab0, ed01, ew, zeros, out1, out2, outf,
                 acc, eblk, wblk, rowsA, rowsB, zbuf, gA, gB):
    c = lax.axis_index("c")
    s = lax.axis_index("s")
    rbase = s * RPT

    def run_layer(tab, out):
        pltpu.sync_copy(zeros.at[pl.ds(rbase, RPT)], acc.at[pl.ds(rbase, RPT)])
        plsc.subcore_barrier()

        def _work(k, km, cur, gcur, nxt, gnxt):
            pltpu.make_async_copy(tab.at[eblk.at[km]], cur, gcur).wait()

            @pl.when(km != BLK - 1)
            def _():
                pltpu.async_copy(tab.at[eblk.at[km + 1]], nxt, gnxt)

            def _grp(g, c2):
                wvec = wblk[km, pl.ds(g * 16, 16)]
                rb = g * 16
                for lane in range(16):
                    w = wvec[lane]
                    r = rb + lane
                    for j in range(H // 16):
                        cur[r, pl.ds(j * 16, 16)] = (
                            cur[r, pl.ds(j * 16, 16)] * w)
                return c2

            lax.fori_loop(0, C // 16, _grp, 0)
            pltpu.sync_copy(cur, acc.at[eblk.at[BLK + km]], add=True)

        def _chunk(k, carry):
            km = lax.rem(k, BLK)

            @pl.when(km == 0)
            def _():
                bidx = s * NBLK + lax.div(k, BLK)
                pltpu.sync_copy(ed01.at[c * (NSUB * NBLK) + bidx], eblk)
                pltpu.sync_copy(ew.at[bidx], wblk)
                pltpu.async_copy(tab.at[eblk.at[0]], rowsA, gA)

            @pl.when(lax.rem(k, 2) == 0)
            def _():
                _work(k, km, rowsA, gA, rowsB, gB)

            @pl.when(lax.rem(k, 2) == 1)
            def _():
                _work(k, km, rowsB, gB, rowsA, gA)

            return carry

        lax.fori_loop(0, NCHUNK_PT, _chunk, 0)
        plsc.subcore_barrier()
        if out is not None:
            pltpu.sync_copy(acc.at[pl.ds(rbase, RPT)],
                            out.at[pl.ds(c * NP + rbase, RPT)])
            plsc.subcore_barrier()

    run_layer(tab0, out1)
    run_layer(out1, out2)
    run_layer(out2, None)

    for t in range(NFCH):
        r0 = rbase + t * FCH
        g0 = c * NP + r0
        pltpu.sync_copy(acc.at[pl.ds(r0, FCH)], zbuf)
        for srcref in (tab0, out1, out2):
            pltpu.sync_copy(srcref.at[pl.ds(g0, FCH)], rowsA.at[pl.ds(0, FCH)])
            last = srcref is out2

            def _add(i, carry):
                for j in range(H // 16):
                    v = (zbuf[i, pl.ds(j * 16, 16)]
                         + rowsA[i, pl.ds(j * 16, 16)])
                    if last:
                        v = v * 0.25
                    zbuf[i, pl.ds(j * 16, 16)] = v
                return carry

            lax.fori_loop(0, FCH, _add, 0)
        pltpu.sync_copy(zbuf, outf.at[pl.ds(g0, FCH)])


def _pack_edges(src, dst):
    sb = src.reshape(NSUB * NBLK, BLK, C)
    db = dst.reshape(NSUB * NBLK, BLK, C)
    return jnp.concatenate([sb, db], axis=1)


def kernel(user_emb, item_emb, edge_index, edge_weight):
    all_emb = jnp.concatenate([user_emb, item_emb], axis=0)
    rpad = jnp.zeros((NP - N, H), jnp.float32)
    tab0 = jnp.concatenate(
        [all_emb[:, :H], rpad, all_emb[:, H:], rpad], axis=0)
    dst = edge_index[0]
    src = edge_index[1]
    pad = E_PAD - E
    zi = jnp.zeros((pad,), jnp.int32)
    src_p = jnp.concatenate([src, zi])
    dst_p = jnp.concatenate([dst, zi])
    w_p = jnp.concatenate([edge_weight, jnp.zeros((pad,), jnp.float32)])
    ed0 = _pack_edges(src_p, dst_p)
    ed1 = _pack_edges(src_p + NP, dst_p)
    ed01 = jnp.concatenate([ed0, ed1], axis=0)
    ew = w_p.reshape(NSUB * NBLK, BLK, C)
    zeros = jnp.zeros((NP, H), jnp.float32)
    _, _, outf = _lightgcn_sc(tab0, ed01, ew, zeros)
    mean = jnp.concatenate([outf[:N], outf[NP:NP + N]], axis=1)
    return (mean[:N_USERS], mean[N_USERS:])


# speedup vs baseline: 3.2435x; 1.0023x over previous
import functools

import jax
import jax.numpy as jnp
from jax import lax
from jax.experimental import pallas as pl
from jax.experimental.pallas import tpu as pltpu
from jax.experimental.pallas import tpu_sc as plsc

N_USERS = 5000
N = 10000
NP = 10240
D = 256
H = 128
E = 160000
NCORE = 2
NSUB = 16
C = 128
NCHUNK_PT = 80
BLK = 8
NBLK = NCHUNK_PT // BLK
EPT = C * NCHUNK_PT
E_PAD = EPT * NSUB
RPT = NP // NSUB
FCH = 64
NFCH = RPT // FCH

_mesh = plsc.VectorSubcoreMesh(core_axis_name="c", subcore_axis_name="s")


@functools.partial(
    pl.kernel,
    out_type=(
        jax.ShapeDtypeStruct((2 * NP, H), jnp.float32),
        jax.ShapeDtypeStruct((2 * NP, H), jnp.float32),
        jax.ShapeDtypeStruct((2 * NP, H), jnp.float32),
    ),
    mesh=_mesh,
    scratch_types=[
        pltpu.VMEM_SHARED((NP, H), jnp.float32),
        pltpu.VMEM((2 * BLK, C), jnp.int32),
        pltpu.VMEM((BLK, C), jnp.float32),
        pltpu.VMEM((C, H), jnp.float32),
        pltpu.VMEM((C, H), jnp.float32),
        pltpu.VMEM((FCH, H), jnp.float32),
        pltpu.SemaphoreType.DMA,
        pltpu.SemaphoreType.DMA,
        pltpu.SemaphoreType.DMA,
        pltpu.SemaphoreType.DMA,
    ],
)
def _lightgcn_sc(tab0, ed01, ew, zeros, out1, out2, outf,
                 acc, eblk, wblk, rowsA, rowsB, zbuf, gA, gB, sA, sB):
    c = lax.axis_index("c")
    s = lax.axis_index("s")
    rbase = s * RPT

    def run_layer(tab, out):
        pltpu.sync_copy(zeros.at[pl.ds(rbase, RPT)], acc.at[pl.ds(rbase, RPT)])
        plsc.subcore_barrier()

        def _work(k, km, cur, gcur, scur, nxt, gnxt, snxt):
            pltpu.make_async_copy(tab.at[eblk.at[km]], cur, gcur).wait()

            @pl.when(km != 0)
            def _():
                pltpu.make_async_copy(nxt, acc.at[eblk.at[BLK + km]],
                                      snxt).wait()

            @pl.when(km != BLK - 1)
            def _():
                pltpu.async_copy(tab.at[eblk.at[km + 1]], nxt, gnxt)

            def _grp(g, c2):
                wvec = wblk[km, pl.ds(g * 16, 16)]
                rb = g * 16
                for lane in range(16):
                    w = wvec[lane]
                    r = rb + lane
                    for j in range(H // 16):
                        cur[r, pl.ds(j * 16, 16)] = (
                            cur[r, pl.ds(j * 16, 16)] * w)
                return c2

            lax.fori_loop(0, C // 16, _grp, 0)
            pltpu.async_copy(cur, acc.at[eblk.at[BLK + km]], scur, add=True)

        def _chunk(k, carry):
            km = lax.rem(k, BLK)

            @pl.when(km == 0)
            def _():
                @pl.when(k != 0)
                def _():
                    pltpu.make_async_copy(rowsB, acc.at[eblk.at[2 * BLK - 1]],
                                          sB).wait()

                bidx = s * NBLK + lax.div(k, BLK)
                pltpu.sync_copy(ed01.at[c * (NSUB * NBLK) + bidx], eblk)
                pltpu.sync_copy(ew.at[bidx], wblk)
                pltpu.async_copy(tab.at[eblk.at[0]], rowsA, gA)

            @pl.when(lax.rem(k, 2) == 0)
            def _():
                _work(k, km, rowsA, gA, sA, rowsB, gB, sB)

            @pl.when(lax.rem(k, 2) == 1)
            def _():
                _work(k, km, rowsB, gB, sB, rowsA, gA, sA)

            return carry

        lax.fori_loop(0, NCHUNK_PT, _chunk, 0)
        pltpu.make_async_copy(rowsB, acc.at[eblk.at[2 * BLK - 1]], sB).wait()
        plsc.subcore_barrier()
        if out is not None:
            pltpu.sync_copy(acc.at[pl.ds(rbase, RPT)],
                            out.at[pl.ds(c * NP + rbase, RPT)])
            plsc.subcore_barrier()

    run_layer(tab0, out1)
    run_layer(out1, out2)
    run_layer(out2, None)

    for t in range(NFCH):
        r0 = rbase + t * FCH
        g0 = c * NP + r0
        pltpu.sync_copy(acc.at[pl.ds(r0, FCH)], zbuf)
        for srcref in (tab0, out1, out2):
            pltpu.sync_copy(srcref.at[pl.ds(g0, FCH)], rowsA.at[pl.ds(0, FCH)])
            last = srcref is out2

            def _add(i, carry):
                for j in range(H // 16):
                    v = (zbuf[i, pl.ds(j * 16, 16)]
                         + rowsA[i, pl.ds(j * 16, 16)])
                    if last:
                        v = v * 0.25
                    zbuf[i, pl.ds(j * 16, 16)] = v
                return carry

            lax.fori_loop(0, FCH, _add, 0)
        pltpu.sync_copy(zbuf, outf.at[pl.ds(g0, FCH)])


def _pack_edges(src, dst):
    sb = src.reshape(NSUB * NBLK, BLK, C)
    db = dst.reshape(NSUB * NBLK, BLK, C)
    return jnp.concatenate([sb, db], axis=1)


def kernel(user_emb, item_emb, edge_index, edge_weight):
    all_emb = jnp.concatenate([user_emb, item_emb], axis=0)
    rpad = jnp.zeros((NP - N, H), jnp.float32)
    tab0 = jnp.concatenate(
        [all_emb[:, :H], rpad, all_emb[:, H:], rpad], axis=0)
    dst = edge_index[0]
    src = edge_index[1]
    pad = E_PAD - E
    zi = jnp.zeros((pad,), jnp.int32)
    src_p = jnp.concatenate([src, zi])
    dst_p = jnp.concatenate([dst, zi])
    w_p = jnp.concatenate([edge_weight, jnp.zeros((pad,), jnp.float32)])
    ed0 = _pack_edges(src_p, dst_p)
    ed1 = _pack_edges(src_p + NP, dst_p)
    ed01 = jnp.concatenate([ed0, ed1], axis=0)
    ew = w_p.reshape(NSUB * NBLK, BLK, C)
    zeros = jnp.zeros((NP, H), jnp.float32)
    _, _, outf = _lightgcn_sc(tab0, ed01, ew, zeros)
    mean = jnp.concatenate([outf[:N], outf[NP:NP + N]], axis=1)
    return (mean[:N_USERS], mean[N_USERS:])
